# baseline (speedup 1.0000x reference)
import jax
import jax.numpy as jnp
from jax import lax
from jax.experimental import pallas as pl
from jax.experimental.pallas import tpu as pltpu
from jax.experimental.pallas import tpu_sc as plsc

N = 10000
E = 320000
D_H = 128
D_OUT = 40

NC = 2
NS = 16
NW = NC * NS
K = 128
CH = 40
NHALF = 2
EPT = NHALF * CH * K
EPAD = NW * EPT
NP = 10240
RPT = NP // NS

_mesh = plsc.VectorSubcoreMesh(core_axis_name="c", subcore_axis_name="s")


def _make_sc_agg(D):
  def body(table, src_p, dst_p, out,
           src_buf, dst_buf, gbuf_a, gbuf_b, sem_a, sem_b, shared):
    c = lax.axis_index("c")
    s = lax.axis_index("s")
    wid = c * NS + s

    def src_idx(j):
      return src_buf.at[pl.ds(j * K, K)]

    def zrow(r, carry):
      for col in range(D // 16):
        gbuf_a[r, pl.ds(col * 16, 16)] = jnp.zeros((16,), jnp.float32)
      return carry

    lax.fori_loop(0, K, zrow, 0)
    for i in range(RPT // K):
      pltpu.async_copy(gbuf_a, shared.at[pl.ds(s * RPT + i * K, K)], sem_a)
    for i in range(RPT // K):
      pltpu.make_async_copy(gbuf_a, shared.at[pl.ds(s * RPT + i * K, K)],
                            sem_a).wait()
    plsc.subcore_barrier()

    for h in range(NHALF):
      pltpu.sync_copy(src_p.at[wid, h], src_buf)
      pltpu.sync_copy(dst_p.at[wid, h], dst_buf)
      pltpu.async_copy(table.at[src_idx(0)], gbuf_a, sem_a)

      def pair(i, carry):
        j0 = 2 * i
        pltpu.make_async_copy(table.at[src_idx(j0)], gbuf_a, sem_a).wait()
        pltpu.async_copy(table.at[src_idx(j0 + 1)], gbuf_b, sem_b)
        pltpu.sync_copy(gbuf_a, shared.at[dst_buf.at[j0]], add=True)
        pltpu.make_async_copy(table.at[src_idx(j0 + 1)], gbuf_b, sem_b).wait()
        pltpu.async_copy(table.at[src_idx(j0 + 2)], gbuf_a, sem_a)
        pltpu.sync_copy(gbuf_b, shared.at[dst_buf.at[j0 + 1]], add=True)
        return carry

      lax.fori_loop(0, (CH - 2) // 2, pair, 0)
      pltpu.make_async_copy(table.at[src_idx(CH - 2)], gbuf_a, sem_a).wait()
      pltpu.async_copy(table.at[src_idx(CH - 1)], gbuf_b, sem_b)
      pltpu.sync_copy(gbuf_a, shared.at[dst_buf.at[CH - 2]], add=True)
      pltpu.make_async_copy(table.at[src_idx(CH - 1)], gbuf_b, sem_b).wait()
      pltpu.sync_copy(gbuf_b, shared.at[dst_buf.at[CH - 1]], add=True)

    plsc.subcore_barrier()

    pltpu.sync_copy(shared.at[pl.ds(s * RPT, RPT)],
                    out.at[c, pl.ds(s * RPT, RPT)])

  scratch = [
      pltpu.VMEM((CH * K,), jnp.int32),
      pltpu.VMEM((CH, K), jnp.int32),
      pltpu.VMEM((K, D), jnp.float32),
      pltpu.VMEM((K, D), jnp.float32),
      pltpu.SemaphoreType.DMA,
      pltpu.SemaphoreType.DMA,
      pltpu.VMEM_SHARED((NP, D), jnp.float32),
  ]

  return pl.kernel(
      body,
      mesh=_mesh,
      out_type=jax.ShapeDtypeStruct((NC, NP, D), jnp.float32),
      scratch_types=scratch,
  )


def _sc_deg_body(dst_p, out, dst_buf, ones_v, zbuf, shared):
  c = lax.axis_index("c")
  s = lax.axis_index("s")
  wid = c * NS + s
  for i in range(K // 16):
    ones_v[pl.ds(i * 16, 16)] = jnp.ones((16,), jnp.float32)

  def zrow(r, carry):
    zbuf[pl.ds(r * 16, 16)] = jnp.zeros((16,), jnp.float32)
    return carry

  lax.fori_loop(0, RPT // 16, zrow, 0)
  pltpu.sync_copy(zbuf, shared.at[pl.ds(s * RPT, RPT)])
  plsc.subcore_barrier()

  for h in range(NHALF):
    pltpu.sync_copy(dst_p.at[wid, h], dst_buf)

    def chunk(j, carry):
      pltpu.sync_copy(ones_v, shared.at[dst_buf.at[j]], add=True)
      return carry

    lax.fori_loop(0, CH, chunk, 0)

  plsc.subcore_barrier()
  pltpu.sync_copy(shared.at[pl.ds(s * RPT, RPT)],
                  out.at[c, pl.ds(s * RPT, RPT)])


_sc_deg = pl.kernel(
    _sc_deg_body,
    mesh=_mesh,
    out_type=jax.ShapeDtypeStruct((NC, NP), jnp.float32),
    scratch_types=[
        pltpu.VMEM((CH, K), jnp.int32),
        pltpu.VMEM((K,), jnp.float32),
        pltpu.VMEM((RPT,), jnp.float32),
        pltpu.VMEM_SHARED((NP,), jnp.float32),
    ],
)


def _tc_mm1(x0_ref, w1_ref, h1_ref):
  h1_ref[...] = jnp.dot(x0_ref[...], w1_ref[...],
                        preferred_element_type=jnp.float32)


def _tc_scale(dp_ref, h1_ref, hp1_ref, dinv_ref):
  deg = dp_ref[0, 0:N] + dp_ref[1, 0:N] + 1.0
  dinv = jnp.where(deg > 0, lax.rsqrt(jnp.maximum(deg, 1e-12)), 0.0)
  dinv_ref[...] = dinv
  hp1_ref[...] = h1_ref[...] * dinv


def _tc_mid(ag_ref, hp1_ref, dinv_ref, b1_ref, g_ref, be_ref, w2_ref,
            x1_ref, hp2_ref):
  dinv = dinv_ref[...]
  hp1 = hp1_ref[...]
  t = dinv * (ag_ref[0, 0:N] + ag_ref[1, 0:N] + hp1) + b1_ref[...]
  mean = jnp.mean(t, axis=0, keepdims=True)
  var = jnp.mean((t - mean) ** 2, axis=0, keepdims=True)
  x1 = g_ref[...] * (t - mean) * lax.rsqrt(var + 1e-5) + be_ref[...]
  x1 = jnp.maximum(x1, 0.0)
  x1_ref[...] = x1
  h2 = jnp.dot(x1, w2_ref[...], preferred_element_type=jnp.float32)
  hp2_ref[...] = h2 * dinv


def _tc_post(ag_ref, hp2_ref, dinv_ref, b2_ref, x1_ref, w3_ref, b3_ref,
             out_ref):
  dinv = dinv_ref[...]
  x2 = dinv * (ag_ref[0, 0:N] + ag_ref[1, 0:N] + hp2_ref[...]) + b2_ref[...]
  x4 = (jnp.dot(x1_ref[...], w3_ref[0:D_H, :],
                preferred_element_type=jnp.float32)
        + jnp.dot(x2, w3_ref[D_H:, :], preferred_element_type=jnp.float32)
        + b3_ref[...])
  m = jnp.max(x4, axis=-1, keepdims=True)
  shifted = x4 - m
  lse = jnp.log(jnp.sum(jnp.exp(shifted), axis=-1, keepdims=True))
  out_ref[...] = shifted - lse


def kernel(x0, edge_index, W1, b1, gamma, beta, W2, b2, W3, b3):
  npad = EPAD - E
  srcf = jnp.concatenate([edge_index[0], jnp.zeros((npad,), jnp.int32)])
  dstf = jnp.concatenate([edge_index[1], jnp.full((npad,), N, jnp.int32)])
  src = srcf.reshape(NW, NHALF, CH * K)
  dst = dstf.reshape(NW, NHALF, CH, K)

  sc_agg = _make_sc_agg(D_H)

  deg_parts = _sc_deg(dst).reshape(NC, NP, 1)

  h1 = pl.pallas_call(
      _tc_mm1,
      out_shape=jax.ShapeDtypeStruct((N, D_H), jnp.float32),
  )(x0, W1)

  hp1, dinv = pl.pallas_call(
      _tc_scale,
      out_shape=[
          jax.ShapeDtypeStruct((N, D_H), jnp.float32),
          jax.ShapeDtypeStruct((N, 1), jnp.float32),
      ],
  )(deg_parts, h1)

  agg1 = sc_agg(hp1, src, dst)

  x1, hp2 = pl.pallas_call(
      _tc_mid,
      out_shape=[
          jax.ShapeDtypeStruct((N, D_H), jnp.float32),
          jax.ShapeDtypeStruct((N, D_H), jnp.float32),
      ],
  )(agg1, hp1, dinv, b1.reshape(1, D_H), gamma.reshape(1, D_H),
    beta.reshape(1, D_H), W2)

  agg2 = sc_agg(hp2, src, dst)

  out = pl.pallas_call(
      _tc_post,
      out_shape=jax.ShapeDtypeStruct((N, D_OUT), jnp.float32),
  )(agg2, hp2, dinv, b2.reshape(1, D_H), x1, W3, b3.reshape(1, D_OUT))

  return out


# speedup vs baseline: 1.0019x; 1.0019x over previous
import jax
import jax.numpy as jnp
from jax import lax
from jax.experimental import pallas as pl
from jax.experimental.pallas import tpu as pltpu
from jax.experimental.pallas import tpu_sc as plsc

N = 10000
E = 320000
D_H = 128
D_OUT = 40

NC = 2
NS = 16
NW = NC * NS
K = 128
CH = 40
NHALF = 2
EPT = NHALF * CH * K
EPAD = NW * EPT
NP = 10240
RPT = NP // NS

_mesh = plsc.VectorSubcoreMesh(core_axis_name="c", subcore_axis_name="s")


def _make_sc_agg(D):
  def body(table, src_p, dst_p, out,
           src_buf, dst_buf, gbuf_a, gbuf_b, sem_a, sem_b, shared):
    c = lax.axis_index("c")
    s = lax.axis_index("s")
    wid = c * NS + s

    def src_idx(j):
      return src_buf.at[pl.ds(j * K, K)]

    def zrow(r, carry):
      for col in range(D // 16):
        gbuf_a[r, pl.ds(col * 16, 16)] = jnp.zeros((16,), jnp.float32)
      return carry

    lax.fori_loop(0, K, zrow, 0)
    for i in range(RPT // K):
      pltpu.async_copy(gbuf_a, shared.at[pl.ds(s * RPT + i * K, K)], sem_a)
    for i in range(RPT // K):
      pltpu.make_async_copy(gbuf_a, shared.at[pl.ds(s * RPT + i * K, K)],
                            sem_a).wait()
    plsc.subcore_barrier()

    for h in range(NHALF):
      pltpu.sync_copy(src_p.at[wid, h], src_buf)
      pltpu.sync_copy(dst_p.at[wid, h], dst_buf)
      pltpu.async_copy(table.at[src_idx(0)], gbuf_a, sem_a)

      def pair(i, carry):
        j0 = 2 * i
        pltpu.make_async_copy(table.at[src_idx(j0)], gbuf_a, sem_a).wait()
        pltpu.async_copy(table.at[src_idx(j0 + 1)], gbuf_b, sem_b)
        pltpu.sync_copy(gbuf_a, shared.at[dst_buf.at[j0]], add=True)
        pltpu.make_async_copy(table.at[src_idx(j0 + 1)], gbuf_b, sem_b).wait()
        pltpu.async_copy(table.at[src_idx(j0 + 2)], gbuf_a, sem_a)
        pltpu.sync_copy(gbuf_b, shared.at[dst_buf.at[j0 + 1]], add=True)
        return carry

      lax.fori_loop(0, (CH - 2) // 2, pair, 0)
      pltpu.make_async_copy(table.at[src_idx(CH - 2)], gbuf_a, sem_a).wait()
      pltpu.async_copy(table.at[src_idx(CH - 1)], gbuf_b, sem_b)
      pltpu.sync_copy(gbuf_a, shared.at[dst_buf.at[CH - 2]], add=True)
      pltpu.make_async_copy(table.at[src_idx(CH - 1)], gbuf_b, sem_b).wait()
      pltpu.sync_copy(gbuf_b, shared.at[dst_buf.at[CH - 1]], add=True)

    plsc.subcore_barrier()

    pltpu.sync_copy(shared.at[pl.ds(s * RPT, RPT)],
                    out.at[c, pl.ds(s * RPT, RPT)])

  scratch = [
      pltpu.VMEM((CH * K,), jnp.int32),
      pltpu.VMEM((CH, K), jnp.int32),
      pltpu.VMEM((K, D), jnp.float32),
      pltpu.VMEM((K, D), jnp.float32),
      pltpu.SemaphoreType.DMA,
      pltpu.SemaphoreType.DMA,
      pltpu.VMEM_SHARED((NP, D), jnp.float32),
  ]

  return pl.kernel(
      body,
      mesh=_mesh,
      out_type=jax.ShapeDtypeStruct((NC, NP, D), jnp.float32),
      scratch_types=scratch,
  )


def _sc_deg_body(dst_p, out, dst_buf, ones_v, zbuf, shared):
  c = lax.axis_index("c")
  s = lax.axis_index("s")
  wid = c * NS + s
  for i in range(K // 16):
    ones_v[pl.ds(i * 16, 16)] = jnp.ones((16,), jnp.float32)

  def zrow(r, carry):
    zbuf[pl.ds(r * 16, 16)] = jnp.zeros((16,), jnp.float32)
    return carry

  lax.fori_loop(0, RPT // 16, zrow, 0)
  pltpu.sync_copy(zbuf, shared.at[pl.ds(s * RPT, RPT)])
  plsc.subcore_barrier()

  for h in range(NHALF):
    pltpu.sync_copy(dst_p.at[wid, h], dst_buf)

    def chunk(j, carry):
      pltpu.sync_copy(ones_v, shared.at[dst_buf.at[j]], add=True)
      return carry

    lax.fori_loop(0, CH, chunk, 0)

  plsc.subcore_barrier()
  pltpu.sync_copy(shared.at[pl.ds(s * RPT, RPT)],
                  out.at[c, pl.ds(s * RPT, RPT)])


_sc_deg = pl.kernel(
    _sc_deg_body,
    mesh=_mesh,
    out_type=jax.ShapeDtypeStruct((NC, NP), jnp.float32),
    scratch_types=[
        pltpu.VMEM((CH, K), jnp.int32),
        pltpu.VMEM((K,), jnp.float32),
        pltpu.VMEM((RPT,), jnp.float32),
        pltpu.VMEM_SHARED((NP,), jnp.float32),
    ],
)


def _tc_mm1(x0_ref, w1_ref, h1_ref):
  h1_ref[...] = jnp.dot(x0_ref[...], w1_ref[...],
                        preferred_element_type=jnp.float32)


def _tc_scale(dp_ref, h1_ref, hp1_ref, dinv_ref):
  deg = dp_ref[0, 0:N] + dp_ref[1, 0:N] + 1.0
  dinv = jnp.where(deg > 0, lax.rsqrt(jnp.maximum(deg, 1e-12)), 0.0)
  dinv_ref[...] = dinv
  hp1_ref[...] = h1_ref[...] * dinv


def _tc_mid(ag_ref, hp1_ref, dinv_ref, b1_ref, g_ref, be_ref, w2_ref,
            x1_ref, hp2_ref):
  dinv = dinv_ref[...]
  hp1 = hp1_ref[...]
  t = dinv * (ag_ref[0, 0:N] + ag_ref[1, 0:N] + hp1) + b1_ref[...]
  mean = jnp.mean(t, axis=0, keepdims=True)
  var = jnp.mean((t - mean) ** 2, axis=0, keepdims=True)
  x1 = g_ref[...] * (t - mean) * lax.rsqrt(var + 1e-5) + be_ref[...]
  x1 = jnp.maximum(x1, 0.0)
  x1_ref[...] = x1
  h2 = jnp.dot(x1, w2_ref[...], preferred_element_type=jnp.float32)
  hp2_ref[...] = h2 * dinv


def _tc_post(ag_ref, hp2_ref, dinv_ref, b2_ref, x1_ref, w3_ref, b3_ref,
             out_ref):
  dinv = dinv_ref[...]
  x2 = dinv * (ag_ref[0, 0:N] + ag_ref[1, 0:N] + hp2_ref[...]) + b2_ref[...]
  x4 = (jnp.dot(x1_ref[...], w3_ref[0:D_H, :],
                preferred_element_type=jnp.float32)
        + jnp.dot(x2, w3_ref[D_H:, :], preferred_element_type=jnp.float32)
        + b3_ref[...])
  m = jnp.max(x4, axis=-1, keepdims=True)
  shifted = x4 - m
  lse = jnp.log(jnp.sum(jnp.exp(shifted), axis=-1, keepdims=True))
  out_ref[...] = shifted - lse


def kernel(x0, edge_index, W1, b1, gamma, beta, W2, b2, W3, b3):
  npad = EPAD - E
  srcf = jnp.concatenate([edge_index[0], jnp.zeros((npad,), jnp.int32)])
  pad_dst = N + jnp.arange(npad, dtype=jnp.int32) % (NP - N)
  dstf = jnp.concatenate([edge_index[1], pad_dst])
  src = srcf.reshape(NW, NHALF, CH * K)
  dst = dstf.reshape(NW, NHALF, CH, K)

  sc_agg = _make_sc_agg(D_H)

  deg_parts = _sc_deg(dst).reshape(NC, NP, 1)

  h1 = pl.pallas_call(
      _tc_mm1,
      out_shape=jax.ShapeDtypeStruct((N, D_H), jnp.float32),
  )(x0, W1)

  hp1, dinv = pl.pallas_call(
      _tc_scale,
      out_shape=[
          jax.ShapeDtypeStruct((N, D_H), jnp.float32),
          jax.ShapeDtypeStruct((N, 1), jnp.float32),
      ],
  )(deg_parts, h1)

  agg1 = sc_agg(hp1, src, dst)

  x1, hp2 = pl.pallas_call(
      _tc_mid,
      out_shape=[
          jax.ShapeDtypeStruct((N, D_H), jnp.float32),
          jax.ShapeDtypeStruct((N, D_H), jnp.float32),
      ],
  )(agg1, hp1, dinv, b1.reshape(1, D_H), gamma.reshape(1, D_H),
    beta.reshape(1, D_H), W2)

  agg2 = sc_agg(hp2, src, dst)

  out = pl.pallas_call(
      _tc_post,
      out_shape=jax.ShapeDtypeStruct((N, D_OUT), jnp.float32),
  )(agg2, hp2, dinv, b2.reshape(1, D_H), x1, W3, b3.reshape(1, D_OUT))

  return out


# speedup vs baseline: 2.7661x; 2.7609x over previous
import jax
import jax.numpy as jnp
from jax import lax
from jax.experimental import pallas as pl
from jax.experimental.pallas import tpu as pltpu
from jax.experimental.pallas import tpu_sc as plsc

N = 10000
E = 320000
D_H = 128
D_OUT = 40

NC = 2
NS = 16
NW = NC * NS
K = 80
NCHUNK = 125
EPT = NCHUNK * K
NP = 10240
RPT = NP // NS

_mesh = plsc.VectorSubcoreMesh(core_axis_name="c", subcore_axis_name="s")


def _make_sc_agg(D):
  def body(table, src_p, dst_p, out,
           src_buf, dst_buf, gbuf_a, gbuf_b, sem_a, sem_b, shared):
    c = lax.axis_index("c")
    s = lax.axis_index("s")
    wid = c * NS + s

    def src_idx(j):
      return src_buf.at[pl.ds(j * K, K)]

    def zrow(r, carry):
      for col in range(D // 16):
        gbuf_a[r, pl.ds(col * 16, 16)] = jnp.zeros((16,), jnp.float32)
      return carry

    lax.fori_loop(0, K, zrow, 0)
    for i in range(RPT // K):
      pltpu.async_copy(gbuf_a, shared.at[pl.ds(s * RPT + i * K, K)], sem_a)
    for i in range(RPT // K):
      pltpu.make_async_copy(gbuf_a, shared.at[pl.ds(s * RPT + i * K, K)],
                            sem_a).wait()
    plsc.subcore_barrier()

    pltpu.sync_copy(src_p.at[wid], src_buf)
    pltpu.sync_copy(dst_p.at[wid], dst_buf)
    pltpu.async_copy(table.at[src_idx(0)], gbuf_a, sem_a)

    def pair(i, carry):
      j0 = 2 * i
      pltpu.make_async_copy(table.at[src_idx(j0)], gbuf_a, sem_a).wait()
      pltpu.async_copy(table.at[src_idx(j0 + 1)], gbuf_b, sem_b)
      pltpu.sync_copy(gbuf_a, shared.at[dst_buf.at[j0]], add=True)
      pltpu.make_async_copy(table.at[src_idx(j0 + 1)], gbuf_b, sem_b).wait()
      pltpu.async_copy(table.at[src_idx(j0 + 2)], gbuf_a, sem_a)
      pltpu.sync_copy(gbuf_b, shared.at[dst_buf.at[j0 + 1]], add=True)
      return carry

    lax.fori_loop(0, (NCHUNK - 1) // 2, pair, 0)
    pltpu.make_async_copy(table.at[src_idx(NCHUNK - 1)], gbuf_a, sem_a).wait()
    pltpu.sync_copy(gbuf_a, shared.at[dst_buf.at[NCHUNK - 1]], add=True)
    plsc.subcore_barrier()

    pltpu.sync_copy(shared.at[pl.ds(s * RPT, RPT)],
                    out.at[c, pl.ds(s * RPT, RPT)])

  scratch = [
      pltpu.VMEM((EPT,), jnp.int32),
      pltpu.VMEM((NCHUNK, K), jnp.int32),
      pltpu.VMEM((K, D), jnp.float32),
      pltpu.VMEM((K, D), jnp.float32),
      pltpu.SemaphoreType.DMA,
      pltpu.SemaphoreType.DMA,
      pltpu.VMEM_SHARED((NP, D), jnp.float32),
  ]

  return pl.kernel(
      body,
      mesh=_mesh,
      out_type=jax.ShapeDtypeStruct((NC, NP, D), jnp.float32),
      scratch_types=scratch,
  )


def _sc_deg_body(dst_p, out, dst_buf, ones_v, zbuf, shared):
  c = lax.axis_index("c")
  s = lax.axis_index("s")
  wid = c * NS + s
  for i in range(K // 16):
    ones_v[pl.ds(i * 16, 16)] = jnp.ones((16,), jnp.float32)

  def zrow(r, carry):
    zbuf[pl.ds(r * 16, 16)] = jnp.zeros((16,), jnp.float32)
    return carry

  lax.fori_loop(0, RPT // 16, zrow, 0)
  pltpu.sync_copy(zbuf, shared.at[pl.ds(s * RPT, RPT)])
  plsc.subcore_barrier()

  pltpu.sync_copy(dst_p.at[wid], dst_buf)

  def chunk(j, carry):
    pltpu.sync_copy(ones_v, shared.at[dst_buf.at[j]], add=True)
    return carry

  lax.fori_loop(0, NCHUNK, chunk, 0)
  plsc.subcore_barrier()
  pltpu.sync_copy(shared.at[pl.ds(s * RPT, RPT)],
                  out.at[c, pl.ds(s * RPT, RPT)])


_sc_deg = pl.kernel(
    _sc_deg_body,
    mesh=_mesh,
    out_type=jax.ShapeDtypeStruct((NC, NP), jnp.float32),
    scratch_types=[
        pltpu.VMEM((NCHUNK, K), jnp.int32),
        pltpu.VMEM((K,), jnp.float32),
        pltpu.VMEM((RPT,), jnp.float32),
        pltpu.VMEM_SHARED((NP,), jnp.float32),
    ],
)


def _tc_mm1(x0_ref, w1_ref, h1_ref):
  h1_ref[...] = jnp.dot(x0_ref[...], w1_ref[...],
                        preferred_element_type=jnp.float32)


def _tc_scale(dp_ref, h1_ref, hp1_ref, dinv_ref):
  deg = dp_ref[0, 0:N] + dp_ref[1, 0:N] + 1.0
  dinv = jnp.where(deg > 0, lax.rsqrt(jnp.maximum(deg, 1e-12)), 0.0)
  dinv_ref[...] = dinv
  hp1_ref[...] = h1_ref[...] * dinv


def _tc_mid(ag_ref, hp1_ref, dinv_ref, b1_ref, g_ref, be_ref, w2_ref,
            x1_ref, hp2_ref):
  dinv = dinv_ref[...]
  hp1 = hp1_ref[...]
  t = dinv * (ag_ref[0, 0:N] + ag_ref[1, 0:N] + hp1) + b1_ref[...]
  mean = jnp.mean(t, axis=0, keepdims=True)
  var = jnp.mean((t - mean) ** 2, axis=0, keepdims=True)
  x1 = g_ref[...] * (t - mean) * lax.rsqrt(var + 1e-5) + be_ref[...]
  x1 = jnp.maximum(x1, 0.0)
  x1_ref[...] = x1
  h2 = jnp.dot(x1, w2_ref[...], preferred_element_type=jnp.float32)
  hp2_ref[...] = h2 * dinv


def _tc_post(ag_ref, hp2_ref, dinv_ref, b2_ref, x1_ref, w3_ref, b3_ref,
             out_ref):
  dinv = dinv_ref[...]
  x2 = dinv * (ag_ref[0, 0:N] + ag_ref[1, 0:N] + hp2_ref[...]) + b2_ref[...]
  x4 = (jnp.dot(x1_ref[...], w3_ref[0:D_H, :],
                preferred_element_type=jnp.float32)
        + jnp.dot(x2, w3_ref[D_H:, :], preferred_element_type=jnp.float32)
        + b3_ref[...])
  m = jnp.max(x4, axis=-1, keepdims=True)
  shifted = x4 - m
  lse = jnp.log(jnp.sum(jnp.exp(shifted), axis=-1, keepdims=True))
  out_ref[...] = shifted - lse


def kernel(x0, edge_index, W1, b1, gamma, beta, W2, b2, W3, b3):
  src = edge_index[0].reshape(NW, EPT)
  dst = edge_index[1].reshape(NW, NCHUNK, K)

  sc_agg = _make_sc_agg(D_H)

  deg_parts = _sc_deg(dst).reshape(NC, NP, 1)

  h1 = pl.pallas_call(
      _tc_mm1,
      out_shape=jax.ShapeDtypeStruct((N, D_H), jnp.float32),
  )(x0, W1)

  hp1, dinv = pl.pallas_call(
      _tc_scale,
      out_shape=[
          jax.ShapeDtypeStruct((N, D_H), jnp.float32),
          jax.ShapeDtypeStruct((N, 1), jnp.float32),
      ],
  )(deg_parts, h1)

  agg1 = sc_agg(hp1, src, dst)

  x1, hp2 = pl.pallas_call(
      _tc_mid,
      out_shape=[
          jax.ShapeDtypeStruct((N, D_H), jnp.float32),
          jax.ShapeDtypeStruct((N, D_H), jnp.float32),
      ],
  )(agg1, hp1, dinv, b1.reshape(1, D_H), gamma.reshape(1, D_H),
    beta.reshape(1, D_H), W2)

  agg2 = sc_agg(hp2, src, dst)

  out = pl.pallas_call(
      _tc_post,
      out_shape=jax.ShapeDtypeStruct((N, D_OUT), jnp.float32),
  )(agg2, hp2, dinv, b2.reshape(1, D_H), x1, W3, b3.reshape(1, D_OUT))

  return out


# speedup vs baseline: 2.8266x; 1.0219x over previous
import jax
import jax.numpy as jnp
from jax import lax
from jax.experimental import pallas as pl
from jax.experimental.pallas import tpu as pltpu
from jax.experimental.pallas import tpu_sc as plsc

N = 10000
E = 320000
D_H = 128
D_OUT = 40

NC = 2
NS = 16
NW = NC * NS
K = 80
NCHUNK = 125
EPT = NCHUNK * K
NP = 10240
RPT = NP // NS

_mesh = plsc.VectorSubcoreMesh(core_axis_name="c", subcore_axis_name="s")


def _make_sc_agg(D):
  def body(table, src_p, dst_p, out,
           src_buf, dst_buf, gbuf_a, gbuf_b, sem_a, sem_b, shared):
    c = lax.axis_index("c")
    s = lax.axis_index("s")
    wid = c * NS + s

    def src_idx(j):
      return src_buf.at[pl.ds(j * K, K)]

    def zrow(r, carry):
      for col in range(D // 16):
        gbuf_a[r, pl.ds(col * 16, 16)] = jnp.zeros((16,), jnp.float32)
      return carry

    lax.fori_loop(0, K, zrow, 0)
    for i in range(RPT // K):
      pltpu.async_copy(gbuf_a, shared.at[pl.ds(s * RPT + i * K, K)], sem_a)
    for i in range(RPT // K):
      pltpu.make_async_copy(gbuf_a, shared.at[pl.ds(s * RPT + i * K, K)],
                            sem_a).wait()
    plsc.subcore_barrier()

    pltpu.sync_copy(src_p.at[wid], src_buf)
    pltpu.sync_copy(dst_p.at[wid], dst_buf)
    pltpu.async_copy(table.at[src_idx(0)], gbuf_a, sem_a)

    def pair(i, carry):
      j0 = 2 * i
      pltpu.make_async_copy(table.at[src_idx(j0)], gbuf_a, sem_a).wait()
      pltpu.async_copy(table.at[src_idx(j0 + 1)], gbuf_b, sem_b)
      pltpu.sync_copy(gbuf_a, shared.at[dst_buf.at[j0]], add=True)
      pltpu.make_async_copy(table.at[src_idx(j0 + 1)], gbuf_b, sem_b).wait()
      pltpu.async_copy(table.at[src_idx(j0 + 2)], gbuf_a, sem_a)
      pltpu.sync_copy(gbuf_b, shared.at[dst_buf.at[j0 + 1]], add=True)
      return carry

    lax.fori_loop(0, (NCHUNK - 1) // 2, pair, 0)
    pltpu.make_async_copy(table.at[src_idx(NCHUNK - 1)], gbuf_a, sem_a).wait()
    pltpu.sync_copy(gbuf_a, shared.at[dst_buf.at[NCHUNK - 1]], add=True)
    plsc.subcore_barrier()

    pltpu.sync_copy(shared.at[pl.ds(s * RPT, RPT)],
                    out.at[c, pl.ds(s * RPT, RPT)])

  scratch = [
      pltpu.VMEM((EPT,), jnp.int32),
      pltpu.VMEM((NCHUNK, K), jnp.int32),
      pltpu.VMEM((K, D), jnp.float32),
      pltpu.VMEM((K, D), jnp.float32),
      pltpu.SemaphoreType.DMA,
      pltpu.SemaphoreType.DMA,
      pltpu.VMEM_SHARED((NP, D), jnp.float32),
  ]

  return pl.kernel(
      body,
      mesh=_mesh,
      out_type=jax.ShapeDtypeStruct((NC, NP, D), jnp.float32),
      scratch_types=scratch,
  )


def _sc_deg_body(dst_p, out, dst_buf, ones_v, zbuf, shared):
  c = lax.axis_index("c")
  s = lax.axis_index("s")
  wid = c * NS + s
  for i in range(K // 16):
    ones_v[pl.ds(i * 16, 16)] = jnp.ones((16,), jnp.float32)

  def zrow(r, carry):
    zbuf[pl.ds(r * 16, 16)] = jnp.zeros((16,), jnp.float32)
    return carry

  lax.fori_loop(0, RPT // 16, zrow, 0)
  pltpu.sync_copy(zbuf, shared.at[pl.ds(s * RPT, RPT)])
  plsc.subcore_barrier()

  pltpu.sync_copy(dst_p.at[wid], dst_buf)

  def chunk(j, carry):
    pltpu.sync_copy(ones_v, shared.at[dst_buf.at[j]], add=True)
    return carry

  lax.fori_loop(0, NCHUNK, chunk, 0)
  plsc.subcore_barrier()
  pltpu.sync_copy(shared.at[pl.ds(s * RPT, RPT)],
                  out.at[c, pl.ds(s * RPT, RPT)])


_sc_deg = pl.kernel(
    _sc_deg_body,
    mesh=_mesh,
    out_type=jax.ShapeDtypeStruct((NC, NP), jnp.float32),
    scratch_types=[
        pltpu.VMEM((NCHUNK, K), jnp.int32),
        pltpu.VMEM((K,), jnp.float32),
        pltpu.VMEM((RPT,), jnp.float32),
        pltpu.VMEM_SHARED((NP,), jnp.float32),
    ],
)


def _tc_mm1(x0_ref, w1_ref, h1_ref):
  h1_ref[...] = jnp.dot(x0_ref[...], w1_ref[...],
                        preferred_element_type=jnp.float32)


def _tc_scale(dp_ref, h1_ref, hp1_ref, dinv_ref):
  deg_row = dp_ref[0:1, 0:N] + dp_ref[1:2, 0:N] + 1.0
  dinv_row = jnp.where(deg_row > 0,
                       lax.rsqrt(jnp.maximum(deg_row, 1e-12)), 0.0)
  dinv = jnp.reshape(dinv_row, (N, 1))
  dinv_ref[...] = dinv
  hp1_ref[...] = h1_ref[...] * dinv


def _tc_mid(ag_ref, hp1_ref, dinv_ref, b1_ref, g_ref, be_ref, w2_ref,
            x1_ref, hp2_ref):
  dinv = dinv_ref[...]
  hp1 = hp1_ref[...]
  t = dinv * (ag_ref[0, 0:N] + ag_ref[1, 0:N] + hp1) + b1_ref[...]
  mean = jnp.mean(t, axis=0, keepdims=True)
  var = jnp.mean((t - mean) ** 2, axis=0, keepdims=True)
  x1 = g_ref[...] * (t - mean) * lax.rsqrt(var + 1e-5) + be_ref[...]
  x1 = jnp.maximum(x1, 0.0)
  x1_ref[...] = x1
  h2 = jnp.dot(x1, w2_ref[...], preferred_element_type=jnp.float32)
  hp2_ref[...] = h2 * dinv


def _tc_post(ag_ref, hp2_ref, dinv_ref, b2_ref, x1_ref, w3_ref, b3_ref,
             out_ref):
  dinv = dinv_ref[...]
  x2 = dinv * (ag_ref[0, 0:N] + ag_ref[1, 0:N] + hp2_ref[...]) + b2_ref[...]
  x4 = (jnp.dot(x1_ref[...], w3_ref[0:D_H, :],
                preferred_element_type=jnp.float32)
        + jnp.dot(x2, w3_ref[D_H:, :], preferred_element_type=jnp.float32)
        + b3_ref[...])
  m = jnp.max(x4, axis=-1, keepdims=True)
  shifted = x4 - m
  lse = jnp.log(jnp.sum(jnp.exp(shifted), axis=-1, keepdims=True))
  out_ref[...] = shifted - lse


def kernel(x0, edge_index, W1, b1, gamma, beta, W2, b2, W3, b3):
  src = edge_index[0].reshape(NW, EPT)
  dst = edge_index[1].reshape(NW, NCHUNK, K)

  sc_agg = _make_sc_agg(D_H)

  deg_parts = _sc_deg(dst)

  h1 = pl.pallas_call(
      _tc_mm1,
      out_shape=jax.ShapeDtypeStruct((N, D_H), jnp.float32),
  )(x0, W1)

  hp1, dinv = pl.pallas_call(
      _tc_scale,
      out_shape=[
          jax.ShapeDtypeStruct((N, D_H), jnp.float32),
          jax.ShapeDtypeStruct((N, 1), jnp.float32),
      ],
  )(deg_parts, h1)

  agg1 = sc_agg(hp1, src, dst)

  x1, hp2 = pl.pallas_call(
      _tc_mid,
      out_shape=[
          jax.ShapeDtypeStruct((N, D_H), jnp.float32),
          jax.ShapeDtypeStruct((N, D_H), jnp.float32),
      ],
  )(agg1, hp1, dinv, b1.reshape(1, D_H), gamma.reshape(1, D_H),
    beta.reshape(1, D_H), W2)

  agg2 = sc_agg(hp2, src, dst)

  out = pl.pallas_call(
      _tc_post,
      out_shape=jax.ShapeDtypeStruct((N, D_OUT), jnp.float32),
  )(agg2, hp2, dinv, b2.reshape(1, D_H), x1, W3, b3.reshape(1, D_OUT))

  return out
